# baseline (speedup 1.0000x reference)
import dataclasses

import jax
import jax.numpy as jnp
from jax import lax
from jax.experimental import pallas as pl
from jax.experimental.pallas import tpu as pltpu
from jax.experimental.pallas import tpu_sc as plsc

_B, _L, _K, _H = 16, 256, 32, 32
_N = _L * _L
_C = 8192
_NCHUNK = _N // _C
_HHALF = _H // 2


def _sc_body(dist_hbm, emb_hbm, out_hbm, emb_v, dist_v, out_v, sem):
    wid = lax.axis_index("s") * 2 + lax.axis_index("c")
    b = wid // 2
    hbase = (wid % 2) * _HHALF
    pltpu.sync_copy(emb_hbm, emb_v)

    @pl.loop(0, _NCHUNK)
    def _chunk(c0):
        pltpu.sync_copy(dist_hbm.at[b, pl.ds(c0 * _C, _C)], dist_v)

        @pl.loop(0, _HHALF)
        def _head(hh):
            h = hbase + hh
            hsplat = jnp.broadcast_to(h, (16,))

            @pl.loop(0, _C, step=16)
            def _vec(i):
                dv = dist_v[pl.ds(i, 16)]
                out_v[pl.ds(i, 16)] = plsc.load_gather(emb_v, [dv, hsplat])

            pltpu.sync_copy(out_v, out_hbm.at[b, h, pl.ds(c0 * _C, _C)])


def kernel(hop_dist, hop_emb):
    B, L, _ = hop_dist.shape
    K, H = hop_emb.shape
    N = L * L
    dist_flat = hop_dist.reshape(B, N)
    mesh = plsc.VectorSubcoreMesh(core_axis_name="c", subcore_axis_name="s")
    cp = pltpu.CompilerParams()
    if "needs_layout_passes" in pltpu.CompilerParams.__dataclass_fields__:
        cp = dataclasses.replace(cp, needs_layout_passes=False)
    k = pl.kernel(
        _sc_body,
        out_type=jax.ShapeDtypeStruct((B, H, N), jnp.float32),
        mesh=mesh,
        compiler_params=cp,
        scratch_types=[
            pltpu.VMEM((K, H), jnp.float32),
            pltpu.VMEM((_C,), jnp.int32),
            pltpu.VMEM((_C,), jnp.float32),
            pltpu.SemaphoreType.DMA,
        ],
    )
    return k(dist_flat, hop_emb).reshape(B, H, L, L)
